# initial kernel scaffold (unmeasured)
import functools

import numpy as np
import jax
import jax.numpy as jnp
from jax import lax
from jax.experimental import pallas as pl
from jax.experimental.pallas import tpu as pltpu

N_DEV = 32
M_CHUNK = 128
WIRE_DTYPE = jnp.float32


def _ring_perm():
    import distributed_mesh_v7x as dm

    mesh = dm.get_mesh("i", world_size=N_DEV)
    coord_to_idx = {}
    for i, d in enumerate(mesh.devices.flat):
        coord_to_idx[tuple(getattr(d, "coords", (i,)))] = i
    cyc = [(0, 0), (0, 1), (0, 2), (0, 3), (1, 3), (1, 2), (1, 1), (2, 1),
           (2, 2), (2, 3), (3, 3), (3, 2), (3, 1), (3, 0), (2, 0), (1, 0)]
    ring_coords = [(0, y, z) for (y, z) in cyc]
    ring_coords += [(1, y, z) for (y, z) in reversed(cyc)]
    try:
        ring = np.array([coord_to_idx[c] for c in ring_coords], dtype=np.int32)
    except KeyError:
        ring = np.arange(N_DEV, dtype=np.int32)
    ring_pos = np.zeros(N_DEV, dtype=np.int32)
    ring_pos[ring] = np.arange(N_DEV, dtype=np.int32)
    return ring, ring_pos


def kernel(x, w_mat):
    ring_np, ring_pos_np = _ring_perm()
    m_total, k = x.shape
    _, n = w_mat.shape
    assert m_total == N_DEV * M_CHUNK

    def body(x_ref, w_ref, out_ref, send_buf, recv_buf, amax_buf,
             send_sems, recv_sems, ag_send_sems, ag_recv_sems, credit_sem):
        ring = jnp.asarray(ring_np)
        ring_pos = jnp.asarray(ring_pos_np)
        my_i = lax.axis_index("i")
        my_r = ring_pos[my_i]
        left = ring[(my_r - 1) % N_DEV]
        right = ring[(my_r + 1) % N_DEV]

        barrier_sem = pltpu.get_barrier_semaphore()
        for nbr in (left, right):
            pl.semaphore_signal(barrier_sem, inc=1, device_id=(nbr,),
                                device_id_type=pl.DeviceIdType.MESH)
        pl.semaphore_wait(barrier_sem, 2)

        def partial(d):
            xs = x_ref[pl.ds(d * M_CHUNK, M_CHUNK), :]
            return jnp.dot(xs, w_ref[:, :], preferred_element_type=jnp.float32)

        def hop(s, val):
            slot = s % 2
            send_buf[slot] = val.astype(WIRE_DTYPE)
            rdma = pltpu.make_async_remote_copy(
                src_ref=send_buf.at[slot],
                dst_ref=recv_buf.at[slot],
                send_sem=send_sems.at[slot],
                recv_sem=recv_sems.at[slot],
                device_id=(right,),
                device_id_type=pl.DeviceIdType.MESH,
            )
            rdma.start()
            rdma.wait()

        hop(0, partial(ring[(my_r - 1) % N_DEV]))

        for s in range(1, N_DEV - 1):
            d = ring[(my_r - 1 - s) % N_DEV]
            val = recv_buf[(s - 1) % 2].astype(jnp.float32) + partial(d)
            if s <= N_DEV - 3:
                pl.semaphore_signal(credit_sem, inc=1, device_id=(left,),
                                    device_id_type=pl.DeviceIdType.MESH)
            if s >= 2:
                pl.semaphore_wait(credit_sem, 1)
            hop(s, val)

        y = recv_buf[(N_DEV - 2) % 2].astype(jnp.float32) + partial(my_i)
        r = jnp.maximum(y, 0.0)

        amax_local = jnp.max(r)
        amax_buf[pl.ds(my_i, 1), :] = jnp.full((1, 128), amax_local,
                                               dtype=jnp.float32)
        for koff in range(1, N_DEV):
            dst = ring[(my_r + koff) % N_DEV]
            rdma = pltpu.make_async_remote_copy(
                src_ref=amax_buf.at[pl.ds(my_i, 1)],
                dst_ref=amax_buf.at[pl.ds(my_i, 1)],
                send_sem=ag_send_sems.at[koff],
                recv_sem=ag_recv_sems.at[my_i],
                device_id=(dst,),
                device_id_type=pl.DeviceIdType.MESH,
            )
            rdma.start()
        for koff in range(1, N_DEV):
            src = ring[(my_r - koff) % N_DEV]
            recv = pltpu.make_async_remote_copy(
                src_ref=amax_buf.at[pl.ds(src, 1)],
                dst_ref=amax_buf.at[pl.ds(src, 1)],
                send_sem=ag_send_sems.at[koff],
                recv_sem=ag_recv_sems.at[src],
                device_id=(src,),
                device_id_type=pl.DeviceIdType.MESH,
            )
            recv.wait_recv()
        for koff in range(1, N_DEV):
            send = pltpu.make_async_remote_copy(
                src_ref=amax_buf.at[pl.ds(my_i, 1)],
                dst_ref=amax_buf.at[pl.ds(my_i, 1)],
                send_sem=ag_send_sems.at[koff],
                recv_sem=ag_recv_sems.at[my_i],
                device_id=(right,),
                device_id_type=pl.DeviceIdType.MESH,
            )
            send.wait_send()

        amax = jnp.max(amax_buf[:, :])
        scale = jnp.maximum(amax, 1e-30) / 127.0
        q = jnp.clip(jnp.round(r / scale), -127.0, 127.0)
        out_ref[:, :] = (q * scale).astype(jnp.float32)

    return pl.pallas_call(
        body,
        out_shape=jax.ShapeDtypeStruct((M_CHUNK, n), jnp.float32),
        in_specs=[
            pl.BlockSpec(memory_space=pltpu.VMEM),
            pl.BlockSpec(memory_space=pltpu.VMEM),
        ],
        out_specs=pl.BlockSpec(memory_space=pltpu.VMEM),
        scratch_shapes=[
            pltpu.VMEM((2, M_CHUNK, n), WIRE_DTYPE),
            pltpu.VMEM((2, M_CHUNK, n), WIRE_DTYPE),
            pltpu.VMEM((N_DEV, 128), jnp.float32),
            pltpu.SemaphoreType.DMA((2,)),
            pltpu.SemaphoreType.DMA((2,)),
            pltpu.SemaphoreType.DMA((N_DEV,)),
            pltpu.SemaphoreType.DMA((N_DEV,)),
            pltpu.SemaphoreType.REGULAR,
        ],
        compiler_params=pltpu.CompilerParams(collective_id=0),
    )(x, w_mat)


# baseline (device time: 1500558 ns/iter reference)
import functools

import numpy as np
import jax
import jax.numpy as jnp
from jax import lax
from jax.experimental import pallas as pl
from jax.experimental.pallas import tpu as pltpu

N_DEV = 32
M_CHUNK = 128
WIRE_DTYPE = jnp.float32


def _ring_perm():
    import distributed_mesh_v7x as dm

    mesh = dm.get_mesh("i", world_size=N_DEV)
    coord_to_idx = {}
    for i, d in enumerate(mesh.devices.flat):
        coord_to_idx[tuple(getattr(d, "coords", (i,)))] = i
    cyc = [(0, 0), (0, 1), (0, 2), (0, 3), (1, 3), (1, 2), (1, 1), (2, 1),
           (2, 2), (2, 3), (3, 3), (3, 2), (3, 1), (3, 0), (2, 0), (1, 0)]
    ring_coords = [(0, y, z) for (y, z) in cyc]
    ring_coords += [(1, y, z) for (y, z) in reversed(cyc)]
    try:
        ring = np.array([coord_to_idx[c] for c in ring_coords], dtype=np.int32)
    except KeyError:
        ring = np.arange(N_DEV, dtype=np.int32)
    ring_pos = np.zeros(N_DEV, dtype=np.int32)
    ring_pos[ring] = np.arange(N_DEV, dtype=np.int32)
    return ring, ring_pos


def kernel(x, w_mat):
    ring_np, ring_pos_np = _ring_perm()
    m_total, k = x.shape
    _, n = w_mat.shape
    assert m_total == N_DEV * M_CHUNK

    def body(ring_ref, ring_pos_ref, x_ref, w_ref, out_ref, send_buf,
             recv_buf, amax_buf, send_sems, recv_sems, ag_send_sems,
             ag_recv_sems, credit_sem):
        def ring(p):
            return ring_ref[p % N_DEV]

        my_i = lax.axis_index("i")
        my_r = ring_pos_ref[my_i]
        left = ring(my_r - 1)
        right = ring(my_r + 1)

        barrier_sem = pltpu.get_barrier_semaphore()
        for nbr in (left, right):
            pl.semaphore_signal(barrier_sem, inc=1, device_id=(nbr,),
                                device_id_type=pl.DeviceIdType.MESH)
        pl.semaphore_wait(barrier_sem, 2)

        def partial(d):
            xs = x_ref[pl.ds(d * M_CHUNK, M_CHUNK), :]
            return jnp.dot(xs, w_ref[:, :], preferred_element_type=jnp.float32)

        def hop(s, val):
            slot = s % 2
            send_buf[slot, :, :] = val.astype(WIRE_DTYPE)
            rdma = pltpu.make_async_remote_copy(
                src_ref=send_buf.at[slot],
                dst_ref=recv_buf.at[slot],
                send_sem=send_sems.at[slot],
                recv_sem=recv_sems.at[slot],
                device_id=(right,),
                device_id_type=pl.DeviceIdType.MESH,
            )
            rdma.start()
            rdma.wait()

        hop(0, partial(ring(my_r - 1)))

        for s in range(1, N_DEV - 1):
            d = ring(my_r - 1 - s)
            val = recv_buf[(s - 1) % 2, :, :].astype(jnp.float32) + partial(d)
            if s <= N_DEV - 3:
                pl.semaphore_signal(credit_sem, inc=1, device_id=(left,),
                                    device_id_type=pl.DeviceIdType.MESH)
            if s >= 2:
                pl.semaphore_wait(credit_sem, 1)
            hop(s, val)

        y = recv_buf[(N_DEV - 2) % 2, :, :].astype(jnp.float32) + partial(my_i)
        r = jnp.maximum(y, 0.0)

        amax_local = jnp.max(r)
        amax_buf[pl.ds(my_i, 1), :] = jnp.full((1, 128), amax_local,
                                               dtype=jnp.float32)
        for koff in range(1, N_DEV):
            dst = ring(my_r + koff)
            rdma = pltpu.make_async_remote_copy(
                src_ref=amax_buf.at[pl.ds(my_i, 1)],
                dst_ref=amax_buf.at[pl.ds(my_i, 1)],
                send_sem=ag_send_sems.at[koff],
                recv_sem=ag_recv_sems.at[my_i],
                device_id=(dst,),
                device_id_type=pl.DeviceIdType.MESH,
            )
            rdma.start()
        for koff in range(1, N_DEV):
            src = ring(my_r - koff)
            recv = pltpu.make_async_remote_copy(
                src_ref=amax_buf.at[pl.ds(src, 1)],
                dst_ref=amax_buf.at[pl.ds(src, 1)],
                send_sem=ag_send_sems.at[koff],
                recv_sem=ag_recv_sems.at[src],
                device_id=(src,),
                device_id_type=pl.DeviceIdType.MESH,
            )
            recv.wait_recv()
        for koff in range(1, N_DEV):
            send = pltpu.make_async_remote_copy(
                src_ref=amax_buf.at[pl.ds(my_i, 1)],
                dst_ref=amax_buf.at[pl.ds(my_i, 1)],
                send_sem=ag_send_sems.at[koff],
                recv_sem=ag_recv_sems.at[my_i],
                device_id=(right,),
                device_id_type=pl.DeviceIdType.MESH,
            )
            send.wait_send()

        amax = jnp.max(amax_buf[:, :])
        scale = jnp.maximum(amax, 1e-30) / 127.0
        q = jnp.clip(jnp.round(r / scale), -127.0, 127.0)
        out_ref[:, :] = (q * scale).astype(jnp.float32)

    return pl.pallas_call(
        body,
        out_shape=jax.ShapeDtypeStruct((M_CHUNK, n), jnp.float32),
        in_specs=[
            pl.BlockSpec(memory_space=pltpu.SMEM),
            pl.BlockSpec(memory_space=pltpu.SMEM),
            pl.BlockSpec(memory_space=pltpu.VMEM),
            pl.BlockSpec(memory_space=pltpu.VMEM),
        ],
        out_specs=pl.BlockSpec(memory_space=pltpu.VMEM),
        scratch_shapes=[
            pltpu.VMEM((2, M_CHUNK, n), WIRE_DTYPE),
            pltpu.VMEM((2, M_CHUNK, n), WIRE_DTYPE),
            pltpu.VMEM((N_DEV, 128), jnp.float32),
            pltpu.SemaphoreType.DMA((2,)),
            pltpu.SemaphoreType.DMA((2,)),
            pltpu.SemaphoreType.DMA((N_DEV,)),
            pltpu.SemaphoreType.DMA((N_DEV,)),
            pltpu.SemaphoreType.REGULAR,
        ],
        compiler_params=pltpu.CompilerParams(collective_id=0),
    )(jnp.asarray(ring_np), jnp.asarray(ring_pos_np), x, w_mat)


# device time: 445617 ns/iter; 3.3674x vs baseline; 3.3674x over previous
import numpy as np
import jax
import jax.numpy as jnp
from jax import lax
from jax.experimental import pallas as pl
from jax.experimental.pallas import tpu as pltpu

N_DEV = 32
M_CHUNK = 128
WIRE_DTYPE = jnp.bfloat16


def _ring_perm():
    import distributed_mesh_v7x as dm

    mesh = dm.get_mesh("i", world_size=N_DEV)
    coord_to_idx = {}
    for i, d in enumerate(mesh.devices.flat):
        coord_to_idx[tuple(getattr(d, "coords", (i,)))] = i
    cyc = [(0, 0), (0, 1), (0, 2), (0, 3), (1, 3), (1, 2), (1, 1), (2, 1),
           (2, 2), (2, 3), (3, 3), (3, 2), (3, 1), (3, 0), (2, 0), (1, 0)]
    ring_coords = [(0, y, z) for (y, z) in cyc]
    ring_coords += [(1, y, z) for (y, z) in reversed(cyc)]
    try:
        ring = np.array([coord_to_idx[c] for c in ring_coords], dtype=np.int32)
    except KeyError:
        ring = np.arange(N_DEV, dtype=np.int32)
    ring_pos = np.zeros(N_DEV, dtype=np.int32)
    ring_pos[ring] = np.arange(N_DEV, dtype=np.int32)
    return ring, ring_pos


def kernel(x, w_mat):
    ring_np, ring_pos_np = _ring_perm()
    m_total, k = x.shape
    _, n = w_mat.shape
    assert m_total == N_DEV * M_CHUNK
    nh = n // 2

    def body(ring_ref, ring_pos_ref, x_ref, w_ref, out_ref,
             cw_send, cw_recv, ccw_send, ccw_recv, amax_buf,
             cw_send_sems, cw_recv_sems, ccw_send_sems, ccw_recv_sems,
             ag_send_sems, ag_recv_sems, cw_credit, ccw_credit):
        def ring(p):
            return ring_ref[p % N_DEV]

        my_i = lax.axis_index("i")
        my_r = ring_pos_ref[my_i]
        left = ring(my_r - 1)
        right = ring(my_r + 1)

        barrier_sem = pltpu.get_barrier_semaphore()
        for nbr in (left, right):
            pl.semaphore_signal(barrier_sem, inc=1, device_id=(nbr,),
                                device_id_type=pl.DeviceIdType.MESH)
        pl.semaphore_wait(barrier_sem, 2)

        def partial_cw(d):
            xs = x_ref[pl.ds(d * M_CHUNK, M_CHUNK), :]
            return jnp.dot(xs, w_ref[:, :nh], preferred_element_type=jnp.float32)

        def partial_ccw(d):
            xs = x_ref[pl.ds(d * M_CHUNK, M_CHUNK), :]
            return jnp.dot(xs, w_ref[:, nh:], preferred_element_type=jnp.float32)

        def cw_rdma(slot):
            return pltpu.make_async_remote_copy(
                src_ref=cw_send.at[slot], dst_ref=cw_recv.at[slot],
                send_sem=cw_send_sems.at[slot], recv_sem=cw_recv_sems.at[slot],
                device_id=(right,), device_id_type=pl.DeviceIdType.MESH)

        def ccw_rdma(slot):
            return pltpu.make_async_remote_copy(
                src_ref=ccw_send.at[slot], dst_ref=ccw_recv.at[slot],
                send_sem=ccw_send_sems.at[slot], recv_sem=ccw_recv_sems.at[slot],
                device_id=(left,), device_id_type=pl.DeviceIdType.MESH)

        cw_send[0, :, :] = partial_cw(ring(my_r - 1)).astype(WIRE_DTYPE)
        ccw_send[0, :, :] = partial_ccw(ring(my_r + 1)).astype(WIRE_DTYPE)
        cw_rdma(0).start()
        ccw_rdma(0).start()

        for s in range(1, N_DEV - 1):
            pslot, slot = (s - 1) % 2, s % 2
            p_cw = partial_cw(ring(my_r - 1 - s))
            p_ccw = partial_ccw(ring(my_r + 1 + s))
            cw_rdma(pslot).wait_recv()
            ccw_rdma(pslot).wait_recv()
            val_cw = cw_recv[pslot, :, :].astype(jnp.float32) + p_cw
            val_ccw = ccw_recv[pslot, :, :].astype(jnp.float32) + p_ccw
            if s <= N_DEV - 3:
                pl.semaphore_signal(cw_credit, inc=1, device_id=(left,),
                                    device_id_type=pl.DeviceIdType.MESH)
                pl.semaphore_signal(ccw_credit, inc=1, device_id=(right,),
                                    device_id_type=pl.DeviceIdType.MESH)
            if s >= 2:
                pl.semaphore_wait(cw_credit, 1)
                pl.semaphore_wait(ccw_credit, 1)
                cw_rdma(slot).wait_send()
                ccw_rdma(slot).wait_send()
            cw_send[slot, :, :] = val_cw.astype(WIRE_DTYPE)
            ccw_send[slot, :, :] = val_ccw.astype(WIRE_DTYPE)
            cw_rdma(slot).start()
            ccw_rdma(slot).start()

        p_cw = partial_cw(my_i)
        p_ccw = partial_ccw(my_i)
        last = (N_DEV - 2) % 2
        cw_rdma(last).wait_recv()
        ccw_rdma(last).wait_recv()
        r_cw = jnp.maximum(cw_recv[last, :, :].astype(jnp.float32) + p_cw, 0.0)
        r_ccw = jnp.maximum(ccw_recv[last, :, :].astype(jnp.float32) + p_ccw, 0.0)

        for slot in (0, 1):
            cw_rdma(slot).wait_send()
            ccw_rdma(slot).wait_send()

        amax_local = jnp.maximum(jnp.max(r_cw), jnp.max(r_ccw))
        amax_buf[pl.ds(my_i, 1), :] = jnp.full((1, 128), amax_local,
                                               dtype=jnp.float32)
        for koff in range(1, N_DEV):
            dst = ring(my_r + koff)
            pltpu.make_async_remote_copy(
                src_ref=amax_buf.at[pl.ds(my_i, 1)],
                dst_ref=amax_buf.at[pl.ds(my_i, 1)],
                send_sem=ag_send_sems.at[koff],
                recv_sem=ag_recv_sems.at[my_i],
                device_id=(dst,), device_id_type=pl.DeviceIdType.MESH,
            ).start()
        for koff in range(1, N_DEV):
            src = ring(my_r - koff)
            pltpu.make_async_remote_copy(
                src_ref=amax_buf.at[pl.ds(src, 1)],
                dst_ref=amax_buf.at[pl.ds(src, 1)],
                send_sem=ag_send_sems.at[koff],
                recv_sem=ag_recv_sems.at[src],
                device_id=(src,), device_id_type=pl.DeviceIdType.MESH,
            ).wait_recv()
        for koff in range(1, N_DEV):
            pltpu.make_async_remote_copy(
                src_ref=amax_buf.at[pl.ds(my_i, 1)],
                dst_ref=amax_buf.at[pl.ds(my_i, 1)],
                send_sem=ag_send_sems.at[koff],
                recv_sem=ag_recv_sems.at[my_i],
                device_id=(right,), device_id_type=pl.DeviceIdType.MESH,
            ).wait_send()

        amax = jnp.max(amax_buf[:, :])
        scale = jnp.maximum(amax, 1e-30) / 127.0
        out_ref[:, :nh] = (jnp.clip(jnp.round(r_cw / scale), -127.0, 127.0)
                           * scale).astype(jnp.float32)
        out_ref[:, nh:] = (jnp.clip(jnp.round(r_ccw / scale), -127.0, 127.0)
                           * scale).astype(jnp.float32)

    return pl.pallas_call(
        body,
        out_shape=jax.ShapeDtypeStruct((M_CHUNK, n), jnp.float32),
        in_specs=[
            pl.BlockSpec(memory_space=pltpu.SMEM),
            pl.BlockSpec(memory_space=pltpu.SMEM),
            pl.BlockSpec(memory_space=pltpu.VMEM),
            pl.BlockSpec(memory_space=pltpu.VMEM),
        ],
        out_specs=pl.BlockSpec(memory_space=pltpu.VMEM),
        scratch_shapes=[
            pltpu.VMEM((2, M_CHUNK, nh), WIRE_DTYPE),
            pltpu.VMEM((2, M_CHUNK, nh), WIRE_DTYPE),
            pltpu.VMEM((2, M_CHUNK, nh), WIRE_DTYPE),
            pltpu.VMEM((2, M_CHUNK, nh), WIRE_DTYPE),
            pltpu.VMEM((N_DEV, 128), jnp.float32),
            pltpu.SemaphoreType.DMA((2,)),
            pltpu.SemaphoreType.DMA((2,)),
            pltpu.SemaphoreType.DMA((2,)),
            pltpu.SemaphoreType.DMA((2,)),
            pltpu.SemaphoreType.DMA((N_DEV,)),
            pltpu.SemaphoreType.DMA((N_DEV,)),
            pltpu.SemaphoreType.REGULAR,
            pltpu.SemaphoreType.REGULAR,
        ],
        compiler_params=pltpu.CompilerParams(collective_id=0),
    )(jnp.asarray(ring_np), jnp.asarray(ring_pos_np), x, w_mat)


# device time: 372081 ns/iter; 4.0329x vs baseline; 1.1976x over previous
import numpy as np
import jax
import jax.numpy as jnp
from jax import lax
from jax.experimental import pallas as pl
from jax.experimental.pallas import tpu as pltpu

N_DEV = 32
M_CHUNK = 128
WIRE_DTYPE = jnp.bfloat16

CW, CCW = 0, 1
STREAMS = ((CW, 0), (CCW, 2), (CW, 1), (CCW, 3))
N_STREAMS = len(STREAMS)


def _ring_perm():
    import distributed_mesh_v7x as dm

    mesh = dm.get_mesh("i", world_size=N_DEV)
    coord_to_idx = {}
    for i, d in enumerate(mesh.devices.flat):
        coord_to_idx[tuple(getattr(d, "coords", (i,)))] = i
    cyc = [(0, 0), (0, 1), (0, 2), (0, 3), (1, 3), (1, 2), (1, 1), (2, 1),
           (2, 2), (2, 3), (3, 3), (3, 2), (3, 1), (3, 0), (2, 0), (1, 0)]
    ring_coords = [(0, y, z) for (y, z) in cyc]
    ring_coords += [(1, y, z) for (y, z) in reversed(cyc)]
    try:
        ring = np.array([coord_to_idx[c] for c in ring_coords], dtype=np.int32)
    except KeyError:
        ring = np.arange(N_DEV, dtype=np.int32)
    ring_pos = np.zeros(N_DEV, dtype=np.int32)
    ring_pos[ring] = np.arange(N_DEV, dtype=np.int32)
    return ring, ring_pos


def kernel(x, w_mat):
    ring_np, ring_pos_np = _ring_perm()
    m_total, k = x.shape
    _, n = w_mat.shape
    assert m_total == N_DEV * M_CHUNK
    nq = n // 4

    def body(ring_ref, ring_pos_ref, x_ref, w_ref, out_ref,
             send_buf, recv_buf, amax_buf, send_sems, recv_sems,
             ag_send_sems, ag_recv_sems, credit_sems):
        def ring(p):
            return ring_ref[p % N_DEV]

        my_i = lax.axis_index("i")
        my_r = ring_pos_ref[my_i]
        left = ring(my_r - 1)
        right = ring(my_r + 1)

        barrier_sem = pltpu.get_barrier_semaphore()
        for nbr in (left, right):
            pl.semaphore_signal(barrier_sem, inc=1, device_id=(nbr,),
                                device_id_type=pl.DeviceIdType.MESH)
        pl.semaphore_wait(barrier_sem, 2)

        def chunk_dev(j, s):
            direction, _ = STREAMS[j]
            return ring(my_r - 1 - s) if direction == CW else ring(my_r + 1 + s)

        def partial(d, j):
            _, q = STREAMS[j]
            xs = x_ref[pl.ds(d * M_CHUNK, M_CHUNK), :]
            return jnp.dot(xs, w_ref[:, q * nq:(q + 1) * nq],
                           preferred_element_type=jnp.float32)

        def rdma(j, slot):
            direction, _ = STREAMS[j]
            return pltpu.make_async_remote_copy(
                src_ref=send_buf.at[j, slot], dst_ref=recv_buf.at[j, slot],
                send_sem=send_sems.at[j, slot], recv_sem=recv_sems.at[j, slot],
                device_id=(right if direction == CW else left,),
                device_id_type=pl.DeviceIdType.MESH)

        def producer_nbr(j):
            direction, _ = STREAMS[j]
            return left if direction == CW else right

        for j in range(N_STREAMS):
            send_buf[j, 0, :, :] = partial(chunk_dev(j, 0), j).astype(WIRE_DTYPE)
            rdma(j, 0).start()

        for s in range(1, N_DEV - 1):
            pslot, slot = (s - 1) % 2, s % 2
            p = [partial(chunk_dev(j, s), j) for j in range(N_STREAMS)]
            for j in range(N_STREAMS):
                rdma(j, pslot).wait_recv()
                val = recv_buf[j, pslot, :, :].astype(jnp.float32) + p[j]
                if s <= N_DEV - 3:
                    pl.semaphore_signal(credit_sems.at[j], inc=1,
                                        device_id=(producer_nbr(j),),
                                        device_id_type=pl.DeviceIdType.MESH)
                if s >= 2:
                    pl.semaphore_wait(credit_sems.at[j], 1)
                    rdma(j, slot).wait_send()
                send_buf[j, slot, :, :] = val.astype(WIRE_DTYPE)
                rdma(j, slot).start()

        last = (N_DEV - 2) % 2
        p = [partial(my_i, j) for j in range(N_STREAMS)]
        r = []
        for j in range(N_STREAMS):
            rdma(j, last).wait_recv()
            r.append(jnp.maximum(
                recv_buf[j, last, :, :].astype(jnp.float32) + p[j], 0.0))

        for j in range(N_STREAMS):
            rdma(j, 0).wait_send()
            rdma(j, 1).wait_send()

        amax_local = jnp.max(jnp.stack([jnp.max(rj) for rj in r]))
        amax_buf[pl.ds(my_i, 1), :] = jnp.full((1, 128), amax_local,
                                               dtype=jnp.float32)
        for koff in range(1, N_DEV):
            dst = ring(my_r + koff)
            pltpu.make_async_remote_copy(
                src_ref=amax_buf.at[pl.ds(my_i, 1)],
                dst_ref=amax_buf.at[pl.ds(my_i, 1)],
                send_sem=ag_send_sems.at[koff],
                recv_sem=ag_recv_sems.at[my_i],
                device_id=(dst,), device_id_type=pl.DeviceIdType.MESH,
            ).start()
        for koff in range(1, N_DEV):
            src = ring(my_r - koff)
            pltpu.make_async_remote_copy(
                src_ref=amax_buf.at[pl.ds(src, 1)],
                dst_ref=amax_buf.at[pl.ds(src, 1)],
                send_sem=ag_send_sems.at[koff],
                recv_sem=ag_recv_sems.at[src],
                device_id=(src,), device_id_type=pl.DeviceIdType.MESH,
            ).wait_recv()
        for koff in range(1, N_DEV):
            pltpu.make_async_remote_copy(
                src_ref=amax_buf.at[pl.ds(my_i, 1)],
                dst_ref=amax_buf.at[pl.ds(my_i, 1)],
                send_sem=ag_send_sems.at[koff],
                recv_sem=ag_recv_sems.at[my_i],
                device_id=(right,), device_id_type=pl.DeviceIdType.MESH,
            ).wait_send()

        amax = jnp.max(amax_buf[:, :])
        scale = jnp.maximum(amax, 1e-30) / 127.0
        for j in range(N_STREAMS):
            _, q = STREAMS[j]
            out_ref[:, q * nq:(q + 1) * nq] = (
                jnp.clip(jnp.round(r[j] / scale), -127.0, 127.0) * scale
            ).astype(jnp.float32)

    return pl.pallas_call(
        body,
        out_shape=jax.ShapeDtypeStruct((M_CHUNK, n), jnp.float32),
        in_specs=[
            pl.BlockSpec(memory_space=pltpu.SMEM),
            pl.BlockSpec(memory_space=pltpu.SMEM),
            pl.BlockSpec(memory_space=pltpu.VMEM),
            pl.BlockSpec(memory_space=pltpu.VMEM),
        ],
        out_specs=pl.BlockSpec(memory_space=pltpu.VMEM),
        scratch_shapes=[
            pltpu.VMEM((N_STREAMS, 2, M_CHUNK, nq), WIRE_DTYPE),
            pltpu.VMEM((N_STREAMS, 2, M_CHUNK, nq), WIRE_DTYPE),
            pltpu.VMEM((N_DEV, 128), jnp.float32),
            pltpu.SemaphoreType.DMA((N_STREAMS, 2)),
            pltpu.SemaphoreType.DMA((N_STREAMS, 2)),
            pltpu.SemaphoreType.DMA((N_DEV,)),
            pltpu.SemaphoreType.DMA((N_DEV,)),
            pltpu.SemaphoreType.REGULAR((N_STREAMS,)),
        ],
        compiler_params=pltpu.CompilerParams(collective_id=0),
    )(jnp.asarray(ring_np), jnp.asarray(ring_pos_np), x, w_mat)
